# Initial kernel scaffold; baseline (speedup 1.0000x reference)
import jax
import jax.numpy as jnp
from jax.experimental import pallas as pl


def kernel(facts, fact_lens, artis, arti_lens, fact_indices, arti_indices, emb, fWih_f, fWhh_f, fbih_f, fbhh_f, fWih_r, fWhh_r, fbih_r, fbhh_r, aWih_f, aWhh_f, abih_f, abhh_f, aWih_r, aWhh_r, abih_r, abhh_r, W1, b1, W2, b2):
    raise NotImplementedError("write your pallas kernel here")


# speedup vs baseline: 14.0855x; 14.0855x over previous
import functools

import jax
import jax.numpy as jnp
from jax import lax
from jax.experimental import pallas as pl
from jax.experimental.pallas import tpu as pltpu
from jax.experimental.pallas import tpu_sc as plsc

FN, FT = 48, 256
AN, AT = 96, 96
D = 256
H = 256
NB = 16

_INTERPRET = False


def _sc_gather(emb, idx_f, idx_a):
    NF = FN * FT
    NA = AN * AT
    info = plsc.get_sparse_core_info()
    NC, NS = info.num_cores, info.num_subcores
    NW = NC * NS
    bf = NF // NW
    ba = NA // NW
    mesh = plsc.VectorSubcoreMesh(core_axis_name="c", subcore_axis_name="s")

    @functools.partial(
        pl.kernel, mesh=mesh,
        out_type=(jax.ShapeDtypeStruct((NF, D), jnp.float32),
                  jax.ShapeDtypeStruct((NA, D), jnp.float32)),
        scratch_types=[
            pltpu.VMEM((bf,), jnp.int32),
            pltpu.VMEM((ba,), jnp.int32),
            pltpu.VMEM((bf, D), jnp.float32),
            pltpu.SemaphoreType.DMA,
        ],
    )
    def k(emb_hbm, idxf_hbm, idxa_hbm, outf_hbm, outa_hbm,
          idxf_v, idxa_v, rows_v, sem):
        wid = lax.axis_index("s") * NC + lax.axis_index("c")
        base_f = wid * bf
        pltpu.sync_copy(idxf_hbm.at[pl.ds(base_f, bf)], idxf_v)
        pltpu.async_copy(emb_hbm.at[idxf_v], rows_v, sem).wait()
        pltpu.sync_copy(rows_v, outf_hbm.at[pl.ds(base_f, bf)])
        base_a = wid * ba
        pltpu.sync_copy(idxa_hbm.at[pl.ds(base_a, ba)], idxa_v)
        pltpu.async_copy(emb_hbm.at[idxa_v], rows_v.at[pl.ds(0, ba)], sem).wait()
        pltpu.sync_copy(rows_v.at[pl.ds(0, ba)], outa_hbm.at[pl.ds(base_a, ba)])

    return k(emb, idx_f, idx_a)


def _lstm_step(x, h, c, acc, W, b, m):
    g = jnp.dot(jnp.concatenate([x, h], axis=1), W,
                preferred_element_type=jnp.float32) + b
    i = jax.nn.sigmoid(g[:, 0:H])
    f = jax.nn.sigmoid(g[:, H:2 * H])
    gg = jnp.tanh(g[:, 2 * H:3 * H])
    o = jax.nn.sigmoid(g[:, 3 * H:4 * H])
    c_new = f * c + i * gg
    h_new = o * jnp.tanh(c_new)
    h2 = jnp.where(m, h_new, h)
    c2 = jnp.where(m, c_new, c)
    acc2 = acc + jnp.where(m, h_new, 0.0)
    return h2, c2, acc2


def _tc_body(ef_ref, ea_ref, lensf_ref, lensa_ref, fidx_ref, aidx_ref,
             WFf_ref, bFf_ref, WFr_ref, bFr_ref,
             WAf_ref, bAf_ref, WAr_ref, bAr_ref,
             W1_ref, b1_ref, W2_ref, b2_ref, out_ref):
    WFf = WFf_ref[...]
    bFf = bFf_ref[...]
    WFr = WFr_ref[...]
    bFr = bFr_ref[...]
    WAf = WAf_ref[...]
    bAf = bAf_ref[...]
    WAr = WAr_ref[...]
    bAr = bAr_ref[...]
    lens_f = lensf_ref[...]
    lens_a = lensa_ref[...]

    def facts_step(s, c):
        hf, cf, af, hr, cr, ar = c
        hf, cf, af = _lstm_step(ef_ref[s], hf, cf, af, WFf, bFf, s < lens_f)
        tr = FT - 1 - s
        hr, cr, ar = _lstm_step(ef_ref[tr], hr, cr, ar, WFr, bFr, tr < lens_f)
        return hf, cf, af, hr, cr, ar

    def artis_step(s, c):
        hf, cf, af, hr, cr, ar = c
        hf, cf, af = _lstm_step(ea_ref[s], hf, cf, af, WAf, bAf, s < lens_a)
        tr = AT - 1 - s
        hr, cr, ar = _lstm_step(ea_ref[tr], hr, cr, ar, WAr, bAr, tr < lens_a)
        return hf, cf, af, hr, cr, ar

    zf = jnp.zeros((FN, H), jnp.float32)
    za = jnp.zeros((AN, H), jnp.float32)
    cf0 = (zf, zf, zf, zf, zf, zf)
    ca0 = (za, za, za, za, za, za)

    def both_step(s, c):
        return facts_step(s, c[0]), artis_step(s, c[1])

    cf1, ca1 = lax.fori_loop(0, AT, both_step, (cf0, ca0))
    cf2 = lax.fori_loop(AT, FT, facts_step, cf1)
    enc_f = jnp.concatenate([cf2[2], cf2[5]], axis=1)
    enc_a = jnp.concatenate([ca1[2], ca1[5]], axis=1)

    iota_f = lax.broadcasted_iota(jnp.int32, (NB, FN), 1)
    iota_a = lax.broadcasted_iota(jnp.int32, (NB, AN), 1)
    fidx = fidx_ref[...]
    aidx = aidx_ref[...]
    Pf = jnp.zeros((NB, FN), jnp.float32)
    for k in range(fidx.shape[1]):
        Pf = Pf + (iota_f == fidx[:, k:k + 1]).astype(jnp.float32)
    Pa = jnp.zeros((NB, AN), jnp.float32)
    for k in range(aidx.shape[1]):
        Pa = Pa + (iota_a == aidx[:, k:k + 1]).astype(jnp.float32)
    sf = jnp.dot(Pf, enc_f, preferred_element_type=jnp.float32)
    sa = jnp.dot(Pa, enc_a, preferred_element_type=jnp.float32)

    x1 = jnp.tanh(jnp.concatenate([sf, sa], axis=1))
    inter = jnp.dot(x1, W1_ref[...], preferred_element_type=jnp.float32) + b1_ref[...]
    out_ref[...] = (jnp.dot(jnp.tanh(inter), W2_ref[...],
                            preferred_element_type=jnp.float32) + b2_ref[...])


def _tc_forward(ef_tm, ea_tm, lens_f, lens_a, fidx, aidx,
                WFf, bFf, WFr, bFr, WAf, bAf, WAr, bAr, W1t, b1, W2t, b2):
    return pl.pallas_call(
        _tc_body,
        out_shape=jax.ShapeDtypeStruct((NB, 12), jnp.float32),
        interpret=_INTERPRET,
    )(ef_tm, ea_tm, lens_f, lens_a, fidx, aidx,
      WFf, bFf, WFr, bFr, WAf, bAf, WAr, bAr, W1t, b1, W2t, b2)


def kernel(facts, fact_lens, artis, arti_lens, fact_indices, arti_indices, emb,
           fWih_f, fWhh_f, fbih_f, fbhh_f, fWih_r, fWhh_r, fbih_r, fbhh_r,
           aWih_f, aWhh_f, abih_f, abhh_f, aWih_r, aWhh_r, abih_r, abhh_r,
           W1, b1, W2, b2):
    idx_f = facts.T.reshape(-1).astype(jnp.int32)
    idx_a = artis.T.reshape(-1).astype(jnp.int32)
    ef_flat, ea_flat = _sc_gather(emb, idx_f, idx_a)
    ef_tm = ef_flat.reshape(FT, FN, D)
    ea_tm = ea_flat.reshape(AT, AN, D)

    WFf = jnp.concatenate([fWih_f.T, fWhh_f.T], axis=0)
    WFr = jnp.concatenate([fWih_r.T, fWhh_r.T], axis=0)
    WAf = jnp.concatenate([aWih_f.T, aWhh_f.T], axis=0)
    WAr = jnp.concatenate([aWih_r.T, aWhh_r.T], axis=0)
    bFf = (fbih_f + fbhh_f)[None, :]
    bFr = (fbih_r + fbhh_r)[None, :]
    bAf = (abih_f + abhh_f)[None, :]
    bAr = (abih_r + abhh_r)[None, :]

    return _tc_forward(
        ef_tm, ea_tm,
        fact_lens.astype(jnp.int32).reshape(FN, 1),
        arti_lens.astype(jnp.int32).reshape(AN, 1),
        fact_indices.astype(jnp.int32), arti_indices.astype(jnp.int32),
        WFf, bFf, WFr, bFr, WAf, bAf, WAr, bAr,
        W1.T, b1[None, :], W2.T, b2[None, :])


# speedup vs baseline: 14.3558x; 1.0192x over previous
import functools

import jax
import jax.numpy as jnp
from jax import lax
from jax.experimental import pallas as pl
from jax.experimental.pallas import tpu as pltpu
from jax.experimental.pallas import tpu_sc as plsc

FN, FT = 48, 256
AN, AT = 96, 96
D = 256
H = 256
NB = 16

_INTERPRET = False


def _sc_gather(emb, idx_f, idx_a):
    NF = FN * FT
    NA = AN * AT
    info = plsc.get_sparse_core_info()
    NC, NS = info.num_cores, info.num_subcores
    NW = NC * NS
    bf = NF // NW
    ba = NA // NW
    mesh = plsc.VectorSubcoreMesh(core_axis_name="c", subcore_axis_name="s")

    @functools.partial(
        pl.kernel, mesh=mesh,
        out_type=(jax.ShapeDtypeStruct((NF, D), jnp.float32),
                  jax.ShapeDtypeStruct((NA, D), jnp.float32)),
        scratch_types=[
            pltpu.VMEM((bf,), jnp.int32),
            pltpu.VMEM((ba,), jnp.int32),
            pltpu.VMEM((bf, D), jnp.float32),
            pltpu.SemaphoreType.DMA,
        ],
    )
    def k(emb_hbm, idxf_hbm, idxa_hbm, outf_hbm, outa_hbm,
          idxf_v, idxa_v, rows_v, sem):
        wid = lax.axis_index("s") * NC + lax.axis_index("c")
        base_f = wid * bf
        pltpu.sync_copy(idxf_hbm.at[pl.ds(base_f, bf)], idxf_v)
        pltpu.async_copy(emb_hbm.at[idxf_v], rows_v, sem).wait()
        pltpu.sync_copy(rows_v, outf_hbm.at[pl.ds(base_f, bf)])
        base_a = wid * ba
        pltpu.sync_copy(idxa_hbm.at[pl.ds(base_a, ba)], idxa_v)
        pltpu.async_copy(emb_hbm.at[idxa_v], rows_v.at[pl.ds(0, ba)], sem).wait()
        pltpu.sync_copy(rows_v.at[pl.ds(0, ba)], outa_hbm.at[pl.ds(base_a, ba)])

    return k(emb, idx_f, idx_a)


def _lstm_step(x, h, c, acc, W, b, m):
    xh = jnp.concatenate([x, h], axis=1).astype(jnp.bfloat16)
    g = jnp.dot(xh, W, preferred_element_type=jnp.float32) + b
    i = jax.nn.sigmoid(g[:, 0:H])
    f = jax.nn.sigmoid(g[:, H:2 * H])
    gg = jnp.tanh(g[:, 2 * H:3 * H])
    o = jax.nn.sigmoid(g[:, 3 * H:4 * H])
    c_new = f * c + i * gg
    h_new = o * jnp.tanh(c_new)
    h2 = jnp.where(m, h_new, h)
    c2 = jnp.where(m, c_new, c)
    acc2 = acc + jnp.where(m, h_new, 0.0)
    return h2, c2, acc2


def _tc_body(ef_ref, ea_ref, lensf_ref, lensa_ref, fidx_ref, aidx_ref,
             WFf_ref, bFf_ref, WFr_ref, bFr_ref,
             WAf_ref, bAf_ref, WAr_ref, bAr_ref,
             W1_ref, b1_ref, W2_ref, b2_ref, out_ref):
    WFf = WFf_ref[...]
    bFf = bFf_ref[...]
    WFr = WFr_ref[...]
    bFr = bFr_ref[...]
    WAf = WAf_ref[...]
    bAf = bAf_ref[...]
    WAr = WAr_ref[...]
    bAr = bAr_ref[...]
    lens_f = lensf_ref[...]
    lens_a = lensa_ref[...]

    def facts_step(s, c):
        hf, cf, af, hr, cr, ar = c
        hf, cf, af = _lstm_step(ef_ref[s], hf, cf, af, WFf, bFf, s < lens_f)
        tr = FT - 1 - s
        hr, cr, ar = _lstm_step(ef_ref[tr], hr, cr, ar, WFr, bFr, tr < lens_f)
        return hf, cf, af, hr, cr, ar

    def artis_step(s, c):
        hf, cf, af, hr, cr, ar = c
        hf, cf, af = _lstm_step(ea_ref[s], hf, cf, af, WAf, bAf, s < lens_a)
        tr = AT - 1 - s
        hr, cr, ar = _lstm_step(ea_ref[tr], hr, cr, ar, WAr, bAr, tr < lens_a)
        return hf, cf, af, hr, cr, ar

    zf = jnp.zeros((FN, H), jnp.float32)
    za = jnp.zeros((AN, H), jnp.float32)
    cf0 = (zf, zf, zf, zf, zf, zf)
    ca0 = (za, za, za, za, za, za)

    def both_step(s, c):
        return facts_step(s, c[0]), artis_step(s, c[1])

    cf1, ca1 = lax.fori_loop(0, AT, both_step, (cf0, ca0))
    cf2 = lax.fori_loop(AT, FT, facts_step, cf1)
    enc_f = jnp.concatenate([cf2[2], cf2[5]], axis=1)
    enc_a = jnp.concatenate([ca1[2], ca1[5]], axis=1)

    iota_f = lax.broadcasted_iota(jnp.int32, (NB, FN), 1)
    iota_a = lax.broadcasted_iota(jnp.int32, (NB, AN), 1)
    fidx = fidx_ref[...]
    aidx = aidx_ref[...]
    Pf = jnp.zeros((NB, FN), jnp.float32)
    for k in range(fidx.shape[1]):
        Pf = Pf + (iota_f == fidx[:, k:k + 1]).astype(jnp.float32)
    Pa = jnp.zeros((NB, AN), jnp.float32)
    for k in range(aidx.shape[1]):
        Pa = Pa + (iota_a == aidx[:, k:k + 1]).astype(jnp.float32)
    sf = jnp.dot(Pf, enc_f, preferred_element_type=jnp.float32)
    sa = jnp.dot(Pa, enc_a, preferred_element_type=jnp.float32)

    x1 = jnp.tanh(jnp.concatenate([sf, sa], axis=1))
    inter = jnp.dot(x1, W1_ref[...], preferred_element_type=jnp.float32) + b1_ref[...]
    out_ref[...] = (jnp.dot(jnp.tanh(inter), W2_ref[...],
                            preferred_element_type=jnp.float32) + b2_ref[...])


def _tc_forward(ef_tm, ea_tm, lens_f, lens_a, fidx, aidx,
                WFf, bFf, WFr, bFr, WAf, bAf, WAr, bAr, W1t, b1, W2t, b2):
    return pl.pallas_call(
        _tc_body,
        out_shape=jax.ShapeDtypeStruct((NB, 12), jnp.float32),
        interpret=_INTERPRET,
    )(ef_tm, ea_tm, lens_f, lens_a, fidx, aidx,
      WFf, bFf, WFr, bFr, WAf, bAf, WAr, bAr, W1t, b1, W2t, b2)


def kernel(facts, fact_lens, artis, arti_lens, fact_indices, arti_indices, emb,
           fWih_f, fWhh_f, fbih_f, fbhh_f, fWih_r, fWhh_r, fbih_r, fbhh_r,
           aWih_f, aWhh_f, abih_f, abhh_f, aWih_r, aWhh_r, abih_r, abhh_r,
           W1, b1, W2, b2):
    idx_f = facts.T.reshape(-1).astype(jnp.int32)
    idx_a = artis.T.reshape(-1).astype(jnp.int32)
    ef_flat, ea_flat = _sc_gather(emb, idx_f, idx_a)
    ef_tm = ef_flat.reshape(FT, FN, D)
    ea_tm = ea_flat.reshape(AT, AN, D)

    WFf = jnp.concatenate([fWih_f.T, fWhh_f.T], axis=0).astype(jnp.bfloat16)
    WFr = jnp.concatenate([fWih_r.T, fWhh_r.T], axis=0).astype(jnp.bfloat16)
    WAf = jnp.concatenate([aWih_f.T, aWhh_f.T], axis=0).astype(jnp.bfloat16)
    WAr = jnp.concatenate([aWih_r.T, aWhh_r.T], axis=0).astype(jnp.bfloat16)
    bFf = (fbih_f + fbhh_f)[None, :]
    bFr = (fbih_r + fbhh_r)[None, :]
    bAf = (abih_f + abhh_f)[None, :]
    bAr = (abih_r + abhh_r)[None, :]

    return _tc_forward(
        ef_tm, ea_tm,
        fact_lens.astype(jnp.int32).reshape(FN, 1),
        arti_lens.astype(jnp.int32).reshape(AN, 1),
        fact_indices.astype(jnp.int32), arti_indices.astype(jnp.int32),
        WFf, bFf, WFr, bFr, WAf, bAf, WAr, bAr,
        W1.T, b1[None, :], W2.T, b2[None, :])
